# Initial kernel scaffold; baseline (speedup 1.0000x reference)
import functools

import jax
import jax.numpy as jnp
from jax import lax
from jax.experimental import pallas as pl
from jax.experimental.pallas import tpu as pltpu
from jax.experimental.pallas import tpu_sc as plsc

ROWS, COLS = 16384, 26
B = ROWS * COLS
D = 64
NC, NS = 2, 16
NW = NC * NS
B_PER_W = B // NW
IW = 128
K = 8
C = K * IW
NCHUNK = B_PER_W // C

_mesh = plsc.VectorSubcoreMesh(core_axis_name="c", subcore_axis_name="s")


@functools.partial(
    pl.kernel,
    mesh=_mesh,
    out_type=jax.ShapeDtypeStruct((B, D), jnp.float32),
    scratch_types=[
        pltpu.VMEM((K, IW), jnp.int32),
        pltpu.VMEM((C, D), jnp.float32),
        pltpu.SemaphoreType.DMA,
    ],
)
def _gather_kernel(ids_hbm, table_hbm, out_hbm, idx_v, rows_v, sem):
    wid = lax.axis_index("s") * NC + lax.axis_index("c")
    base_row = wid * (B_PER_W // IW)

    def body(c, carry):
        row_off = base_row + c * K
        pltpu.sync_copy(ids_hbm.at[pl.ds(row_off, K)], idx_v)
        copies = []
        for j in range(K):
            copies.append(
                pltpu.async_copy(
                    table_hbm.at[idx_v.at[j]],
                    rows_v.at[pl.ds(j * IW, IW)],
                    sem,
                )
            )
        for cp in copies:
            cp.wait()
        pltpu.sync_copy(rows_v, out_hbm.at[pl.ds(row_off * IW, C)])
        return carry

    lax.fori_loop(0, NCHUNK, body, 0)


def kernel(ids, embs, pad):
    del pad
    flat = ids.reshape(B // IW, IW).astype(jnp.int32)
    out = _gather_kernel(flat, embs)
    return out.reshape(ROWS, COLS, D)


# speedup vs baseline: 1.2364x; 1.2364x over previous
import functools

import jax
import jax.numpy as jnp
from jax import lax
from jax.experimental import pallas as pl
from jax.experimental.pallas import tpu as pltpu
from jax.experimental.pallas import tpu_sc as plsc

ROWS, COLS = 16384, 26
B = ROWS * COLS
D = 64
NC, NS = 2, 16
NW = NC * NS
B_PER_W = B // NW
IW = 128
K = 8
C = K * IW
NCHUNK = B_PER_W // C

_mesh = plsc.VectorSubcoreMesh(core_axis_name="c", subcore_axis_name="s")


@functools.partial(
    pl.kernel,
    mesh=_mesh,
    compiler_params=pltpu.CompilerParams(use_tc_tiling_on_sc=False),
    out_type=jax.ShapeDtypeStruct((B, D), jnp.float32),
    scratch_types=[
        pltpu.VMEM((K, IW), jnp.int32),
        pltpu.VMEM((C, D), jnp.float32),
        pltpu.SemaphoreType.DMA,
    ],
)
def _gather_kernel(ids_hbm, table_hbm, out_hbm, idx_v, rows_v, sem):
    wid = lax.axis_index("s") * NC + lax.axis_index("c")
    base_row = wid * (B_PER_W // IW)

    def body(c, carry):
        row_off = base_row + c * K
        pltpu.sync_copy(ids_hbm.at[pl.ds(row_off, K)], idx_v)
        copies = []
        for j in range(K):
            copies.append(
                pltpu.async_copy(
                    table_hbm.at[idx_v.at[j]],
                    rows_v.at[pl.ds(j * IW, IW)],
                    sem,
                )
            )
        for cp in copies:
            cp.wait()
        pltpu.sync_copy(rows_v, out_hbm.at[pl.ds(row_off * IW, C)])
        return carry

    lax.fori_loop(0, NCHUNK, body, 0)


def kernel(ids, embs, pad):
    del pad
    flat = ids.reshape(B // IW, IW).astype(jnp.int32)
    out = _gather_kernel(flat, embs)
    return out.reshape(ROWS, COLS, D)


# speedup vs baseline: 1.2528x; 1.0132x over previous
import functools

import jax
import jax.numpy as jnp
from jax import lax
from jax.experimental import pallas as pl
from jax.experimental.pallas import tpu as pltpu
from jax.experimental.pallas import tpu_sc as plsc

ROWS, COLS = 16384, 26
B = ROWS * COLS
D = 64
NC, NS = 2, 16
NW = NC * NS
B_PER_W = B // NW
IW = 128
KCH = 4
C = KCH * IW
NCH = B_PER_W // C
IROWS = B_PER_W // IW

_mesh = plsc.VectorSubcoreMesh(core_axis_name="c", subcore_axis_name="s")


@functools.partial(
    pl.kernel,
    mesh=_mesh,
    compiler_params=pltpu.CompilerParams(use_tc_tiling_on_sc=False),
    out_type=jax.ShapeDtypeStruct((B, D), jnp.float32),
    scratch_types=[
        pltpu.VMEM((IROWS, IW), jnp.int32),
        pltpu.VMEM((2, C, D), jnp.float32),
        pltpu.SemaphoreType.DMA,
        pltpu.SemaphoreType.DMA,
    ],
)
def _gather_kernel(ids_hbm, table_hbm, out_hbm, idx_v, rows_v, sem_g, sem_s):
    wid = lax.axis_index("s") * NC + lax.axis_index("c")
    base = wid * B_PER_W

    pltpu.sync_copy(ids_hbm.at[pl.ds(wid * IROWS, IROWS)], idx_v)

    def fire_gathers(g, b):
        for j in range(KCH):
            pltpu.async_copy(
                table_hbm.at[idx_v.at[g * KCH + j]],
                rows_v.at[b].at[pl.ds(j * IW, IW)],
                sem_g,
            )

    def wait_gathers(b):
        pltpu.make_async_copy(
            out_hbm.at[pl.ds(0, C)], rows_v.at[b], sem_g
        ).wait()

    def store(g, b):
        return pltpu.async_copy(
            rows_v.at[b], out_hbm.at[pl.ds(base + g * C, C)], sem_s
        )

    fire_gathers(0, 0)
    fire_gathers(1, 1)

    def body(p, carry):
        for b in range(2):
            g = 2 * p + b
            wait_gathers(b)
            store(g, b).wait()
            fire_gathers(g + 2, b)
        return carry

    lax.fori_loop(0, (NCH - 2) // 2, body, 0)

    for b in range(2):
        g = NCH - 2 + b
        wait_gathers(b)
        store(g, b).wait()


def kernel(ids, embs, pad):
    del pad
    flat = ids.reshape(B // IW, IW).astype(jnp.int32)
    out = _gather_kernel(flat, embs)
    return out.reshape(ROWS, COLS, D)


# speedup vs baseline: 1.2926x; 1.0318x over previous
import functools

import jax
import jax.numpy as jnp
from jax import lax
from jax.experimental import pallas as pl
from jax.experimental.pallas import tpu as pltpu
from jax.experimental.pallas import tpu_sc as plsc

ROWS, COLS = 16384, 26
B = ROWS * COLS
D = 64
NC, NS = 2, 16
NW = NC * NS
IW = 128
RB = 1024
NTASK = COLS * (ROWS // RB)
T_PER_W = NTASK // NW
C = 512
KCH = C // IW
NCH = T_PER_W * (RB // C)

_mesh = plsc.VectorSubcoreMesh(core_axis_name="c", subcore_axis_name="s")


@functools.partial(
    pl.kernel,
    mesh=_mesh,
    compiler_params=pltpu.CompilerParams(use_tc_tiling_on_sc=False),
    out_type=jax.ShapeDtypeStruct((COLS, ROWS, D), jnp.float32),
    scratch_types=[
        pltpu.VMEM((T_PER_W, RB // IW, IW), jnp.int32),
        pltpu.VMEM((2, C, D), jnp.float32),
        pltpu.SemaphoreType.DMA,
        pltpu.SemaphoreType.DMA,
    ],
)
def _gather_kernel(ids_hbm, table_hbm, out_hbm, idx_v, rows_v, sem_g, sem_s):
    wid = lax.axis_index("s") * NC + lax.axis_index("c")
    t0 = wid * T_PER_W

    for t in range(T_PER_W):
        pltpu.sync_copy(
            ids_hbm.at[pl.ds((t0 + t) * (RB // IW), RB // IW)], idx_v.at[t]
        )

    def fire_gathers(g, b):
        t = g // 2
        h = g % 2
        for j in range(KCH):
            pltpu.async_copy(
                table_hbm.at[idx_v.at[t].at[h * KCH + j]],
                rows_v.at[b].at[pl.ds(j * IW, IW)],
                sem_g,
            )

    def wait_gathers(b):
        pltpu.make_async_copy(
            out_hbm.at[0].at[pl.ds(0, C)], rows_v.at[b], sem_g
        ).wait()

    def store(g, b):
        task = t0 + g // 2
        col = task // (ROWS // RB)
        r_off = (task % (ROWS // RB)) * RB + (g % 2) * C
        return pltpu.async_copy(
            rows_v.at[b], out_hbm.at[col].at[pl.ds(r_off, C)], sem_s
        )

    fire_gathers(0, 0)
    fire_gathers(1, 1)

    def body(p, carry):
        for b in range(2):
            g = 2 * p + b
            wait_gathers(b)
            store(g, b).wait()
            fire_gathers(g + 2, b)
        return carry

    lax.fori_loop(0, (NCH - 2) // 2, body, 0)

    for b in range(2):
        g = NCH - 2 + b
        wait_gathers(b)
        store(g, b).wait()


def kernel(ids, embs, pad):
    del pad
    ids_cm = ids.T.reshape(B // IW, IW).astype(jnp.int32)
    out_cm = _gather_kernel(ids_cm, embs)
    return out_cm.transpose(1, 0, 2)


# speedup vs baseline: 1.3268x; 1.0264x over previous
import functools

import jax
import jax.numpy as jnp
from jax import lax
from jax.experimental import pallas as pl
from jax.experimental.pallas import tpu as pltpu
from jax.experimental.pallas import tpu_sc as plsc

ROWS, COLS = 16384, 26
B = ROWS * COLS
D = 64
DP = 128
NC, NS = 2, 16
NW = NC * NS
IW = 128
RB = 1024
NTASK = COLS * (ROWS // RB)
T_PER_W = NTASK // NW
C = 512
KCH = C // IW
NCH = T_PER_W * (RB // C)

_mesh = plsc.VectorSubcoreMesh(core_axis_name="c", subcore_axis_name="s")


@functools.partial(
    pl.kernel,
    mesh=_mesh,
    compiler_params=pltpu.CompilerParams(use_tc_tiling_on_sc=False),
    out_type=jax.ShapeDtypeStruct((COLS, ROWS, DP), jnp.float32),
    scratch_types=[
        pltpu.VMEM((T_PER_W, RB // IW, IW), jnp.int32),
        pltpu.VMEM((2, C, D), jnp.float32),
        pltpu.SemaphoreType.DMA,
        pltpu.SemaphoreType.DMA,
    ],
)
def _gather_kernel(ids_hbm, table_hbm, out_hbm, idx_v, rows_v, sem_g, sem_s):
    wid = lax.axis_index("s") * NC + lax.axis_index("c")
    t0 = wid * T_PER_W

    for t in range(T_PER_W):
        pltpu.sync_copy(
            ids_hbm.at[pl.ds((t0 + t) * (RB // IW), RB // IW)], idx_v.at[t]
        )

    def fire_gathers(g, b):
        t = g // 2
        h = g % 2
        for j in range(KCH):
            pltpu.async_copy(
                table_hbm.at[idx_v.at[t].at[h * KCH + j]],
                rows_v.at[b].at[pl.ds(j * IW, IW)],
                sem_g,
            )

    def wait_gathers(b):
        pltpu.make_async_copy(
            out_hbm.at[0].at[pl.ds(0, C), pl.ds(0, D)], rows_v.at[b], sem_g
        ).wait()

    def store(g, b):
        task = t0 + g // 2
        col = task // (ROWS // RB)
        r_off = (task % (ROWS // RB)) * RB + (g % 2) * C
        return pltpu.async_copy(
            rows_v.at[b],
            out_hbm.at[col].at[pl.ds(r_off, C), pl.ds(0, D)],
            sem_s,
        )

    fire_gathers(0, 0)
    fire_gathers(1, 1)

    def body(p, carry):
        for b in range(2):
            g = 2 * p + b
            wait_gathers(b)
            store(g, b).wait()
            fire_gathers(g + 2, b)
        return carry

    lax.fori_loop(0, (NCH - 2) // 2, body, 0)

    for b in range(2):
        g = NCH - 2 + b
        wait_gathers(b)
        store(g, b).wait()


def kernel(ids, embs, pad):
    del pad
    ids_cm = ids.T.reshape(B // IW, IW).astype(jnp.int32)
    out_cm = _gather_kernel(ids_cm, embs)
    return out_cm[:, :, :D].transpose(1, 0, 2)


# speedup vs baseline: 1.5628x; 1.1778x over previous
import functools

import jax
import jax.numpy as jnp
from jax import lax
from jax.experimental import pallas as pl
from jax.experimental.pallas import tpu as pltpu
from jax.experimental.pallas import tpu_sc as plsc

ROWS, COLS = 16384, 26
B = ROWS * COLS
D = 64
DP = 128
NC, NS = 2, 16
NW = NC * NS
IW = 128
RB = 1024
NTASK = COLS * (ROWS // RB)
T_PER_W = NTASK // NW
C = 512
KCH = C // IW
NCH = T_PER_W * (RB // C)

_mesh = plsc.VectorSubcoreMesh(core_axis_name="c", subcore_axis_name="s")


@functools.partial(
    pl.kernel,
    mesh=_mesh,
    compiler_params=pltpu.CompilerParams(use_tc_tiling_on_sc=False),
    out_type=jax.ShapeDtypeStruct((COLS, ROWS, DP), jnp.float32),
    scratch_types=[
        pltpu.VMEM((T_PER_W, RB // IW, IW), jnp.int32),
        pltpu.VMEM((2, C, D), jnp.float32),
        pltpu.SemaphoreType.DMA,
        pltpu.SemaphoreType.DMA,
    ],
)
def _gather_kernel(ids_hbm, table_hbm, out_hbm, idx_v, rows_v, sem_g, sem_s):
    wid = lax.axis_index("s") * NC + lax.axis_index("c")
    t0 = wid * T_PER_W

    for t in range(T_PER_W):
        pltpu.sync_copy(
            ids_hbm.at[pl.ds((t0 + t) * (RB // IW), RB // IW)], idx_v.at[t]
        )

    def fire_gathers(g, b):
        t = g // 2
        h = g % 2
        for j in range(KCH):
            pltpu.async_copy(
                table_hbm.at[idx_v.at[t].at[h * KCH + j]],
                rows_v.at[b].at[pl.ds(j * IW, IW)],
                sem_g,
            )

    def wait_gathers(b):
        pltpu.make_async_copy(
            out_hbm.at[0].at[pl.ds(0, C), pl.ds(0, D)], rows_v.at[b], sem_g
        ).wait()

    def store(g, b):
        task = t0 + g // 2
        col = task // (ROWS // RB)
        r_off = (task % (ROWS // RB)) * RB + (g % 2) * C
        return pltpu.async_copy(
            rows_v.at[b],
            out_hbm.at[col].at[pl.ds(r_off, C), pl.ds(0, D)],
            sem_s,
        )

    fire_gathers(0, 0)
    fire_gathers(1, 1)

    def body(p, carry):
        for b in range(2):
            g = 2 * p + b
            wait_gathers(b)
            store(g, b).wait()
            fire_gathers(g + 2, b)
        return carry

    lax.fori_loop(0, (NCH - 2) // 2, body, 0)

    for b in range(2):
        g = NCH - 2 + b
        wait_gathers(b)
        store(g, b).wait()


def kernel(ids, embs, pad):
    del pad
    ids_cm = ids.T.reshape(B // IW, IW).astype(jnp.int32)
    out_cm = _gather_kernel(ids_cm, embs)
    return out_cm.transpose(1, 0, 2)[:, :, :D]


# speedup vs baseline: 1.8898x; 1.2092x over previous
import functools

import jax
import jax.numpy as jnp
from jax import lax
from jax.experimental import pallas as pl
from jax.experimental.pallas import tpu as pltpu
from jax.experimental.pallas import tpu_sc as plsc

ROWS, COLS = 16384, 26
B = ROWS * COLS
D = 64
DP = 128
NC, NS = 2, 16
NW = NC * NS
IW = 128
RB = 1024
NTASK = COLS * (ROWS // RB)
T_PER_W = NTASK // NW
C = 512
KCH = C // IW
NCH = T_PER_W * (RB // C)

_mesh = plsc.VectorSubcoreMesh(core_axis_name="c", subcore_axis_name="s")

VOCAB = 1000000
HV = VOCAB // 2
_TCB = 1024


_NS = (HV + _TCB - 1) // _TCB
_TAIL = HV - (_NS - 1) * _TCB
_BW = _TCB + 128
_BSK = HV % 128
_VA = VOCAB - D


def _tc_transpose_body(x_hbm, t64_ref, o_ref, xa, xb, sem):
    j = pl.program_id(0)

    def load(slot, jj):
        return pltpu.make_async_copy(
            x_hbm.at[:, pl.ds(jj * _TCB, _TCB)], xa.at[slot], sem
        ), pltpu.make_async_copy(
            x_hbm.at[:, pl.ds(HV - _BSK + jj * _TCB, _BW)], xb.at[slot], sem
        )

    def load_tail(slot):
        return pltpu.make_async_copy(
            x_hbm.at[:, pl.ds((_NS - 1) * _TCB, 384)],
            xa.at[slot].at[:, pl.ds(0, 384)], sem
        ), pltpu.make_async_copy(
            x_hbm.at[:, pl.ds(_VA - 256, 256)],
            xb.at[slot].at[:, pl.ds(0, 256)], sem
        )

    def start(pair):
        pair[0].start()
        pair[1].start()

    def wait(pair):
        pair[0].wait()
        pair[1].wait()

    @pl.when(j == 0)
    def _prime():
        start(load(0, 0))

    @pl.when(j + 1 < _NS - 1)
    def _prefetch():
        start(load((j + 1) % 2, j + 1))

    @pl.when(j + 1 == _NS - 1)
    def _prefetch_tail():
        start(load_tail((j + 1) % 2))

    ident = jnp.eye(D, dtype=jnp.float32)

    def xpose(x):
        return lax.dot_general(
            x, ident, (((0,), (0,)), ((), ())),
            preferred_element_type=jnp.float32,
        )

    @pl.when(j < _NS - 1)
    def _steady():
        wait(load(j % 2, j))
        bt = xpose(xb[j % 2])
        o_ref[...] = jnp.concatenate(
            [
                xpose(xa[j % 2]),
                lax.slice(bt, (_BSK, 0), (_BSK + _TCB, D)),
            ],
            axis=1,
        )

    @pl.when(j == _NS - 1)
    def _tail():
        wait(load_tail(j % 2))
        at = xpose(xa[j % 2, :, :384])
        bt = xpose(xb[j % 2, :, :256])
        t64t = xpose(t64_ref[...])
        bhalf = jnp.concatenate(
            [lax.slice(bt, (_BSK, 0), (256, D)), t64t], axis=0
        )
        o_ref[pl.ds(0, _TAIL), :] = jnp.concatenate(
            [lax.slice(at, (0, 0), (_TAIL, D)), bhalf], axis=1
        )


_tc_transpose = pl.pallas_call(
    _tc_transpose_body,
    grid=(_NS,),
    in_specs=[
        pl.BlockSpec(memory_space=pl.ANY),
        pl.BlockSpec((D, D), lambda j: (0, 0)),
    ],
    out_specs=pl.BlockSpec((_TCB, 2 * D), lambda j: (j, 0)),
    out_shape=jax.ShapeDtypeStruct((HV, 2 * D), jnp.float32),
    scratch_shapes=[
        pltpu.VMEM((2, D, _TCB), jnp.float32),
        pltpu.VMEM((2, D, _BW), jnp.float32),
        pltpu.SemaphoreType.DMA,
    ],
)


@functools.partial(
    pl.kernel,
    mesh=_mesh,
    compiler_params=pltpu.CompilerParams(use_tc_tiling_on_sc=False),
    out_type=jax.ShapeDtypeStruct((COLS, ROWS, DP), jnp.float32),
    scratch_types=[
        pltpu.VMEM((T_PER_W, RB // IW, IW), jnp.int32),
        pltpu.VMEM((2, C, D), jnp.float32),
        pltpu.SemaphoreType.DMA,
        pltpu.SemaphoreType.DMA,
    ],
)
def _gather_kernel(ids_hbm, table_hbm, out_hbm, idx_v, rows_v, sem_g, sem_s):
    wid = lax.axis_index("s") * NC + lax.axis_index("c")
    t0 = wid * T_PER_W

    for t in range(T_PER_W):
        pltpu.sync_copy(
            ids_hbm.at[pl.ds((t0 + t) * (RB // IW), RB // IW)], idx_v.at[t]
        )

    def remap(r, carry):
        row = idx_v.at[r // (RB // IW)].at[r % (RB // IW)]
        for u in range(IW // 16):
            x = row[pl.ds(u * 16, 16)]
            row[pl.ds(u * 16, 16)] = 2 * x - jnp.where(
                x >= HV, jnp.int32(2 * HV - 1), jnp.int32(0)
            )
        return carry

    lax.fori_loop(0, T_PER_W * (RB // IW), remap, 0)

    def fire_gathers(g, b):
        t = g // 2
        h = g % 2
        for j in range(KCH):
            pltpu.async_copy(
                table_hbm.at[idx_v.at[t].at[h * KCH + j]],
                rows_v.at[b].at[pl.ds(j * IW, IW)],
                sem_g,
            )

    def wait_gathers(b):
        pltpu.make_async_copy(
            out_hbm.at[0].at[pl.ds(0, C), pl.ds(0, D)], rows_v.at[b], sem_g
        ).wait()

    def store(g, b):
        task = t0 + g // 2
        col = task // (ROWS // RB)
        r_off = (task % (ROWS // RB)) * RB + (g % 2) * C
        return pltpu.async_copy(
            rows_v.at[b],
            out_hbm.at[col].at[pl.ds(r_off, C), pl.ds(0, D)],
            sem_s,
        )

    fire_gathers(0, 0)
    fire_gathers(1, 1)

    def body(p, carry):
        for b in range(2):
            g = 2 * p + b
            wait_gathers(b)
            store(g, b).wait()
            fire_gathers(g + 2, b)
        return carry

    lax.fori_loop(0, (NCH - 2) // 2, body, 0)

    for b in range(2):
        g = NCH - 2 + b
        wait_gathers(b)
        store(g, b).wait()


def kernel(ids, embs, pad):
    del pad
    ids_cm = ids.T.reshape(B // IW, IW).astype(jnp.int32)
    embs_t = embs.T
    table_rm = _tc_transpose(embs_t, embs_t[:, _VA:]).reshape(VOCAB, D)
    out_cm = _gather_kernel(ids_cm, table_rm)
    return out_cm.transpose(1, 0, 2)[:, :, :D]


# speedup vs baseline: 1.9028x; 1.0069x over previous
import functools

import jax
import jax.numpy as jnp
from jax import lax
from jax.experimental import pallas as pl
from jax.experimental.pallas import tpu as pltpu
from jax.experimental.pallas import tpu_sc as plsc

ROWS, COLS = 16384, 26
B = ROWS * COLS
D = 64
DP = 128
NC, NS = 2, 16
NW = NC * NS
IW = 128
RB = 1024
NTASK = COLS * (ROWS // RB)
T_PER_W = NTASK // NW
C = 512
KCH = C // IW
NCH = T_PER_W * (RB // C)

_mesh = plsc.VectorSubcoreMesh(core_axis_name="c", subcore_axis_name="s")

VOCAB = 1000000
HV = VOCAB // 2
_TCB = 1024


_NS = (HV + _TCB - 1) // _TCB
_TAIL = HV - (_NS - 1) * _TCB
_BW = _TCB + 128
_BSK = HV % 128
_VA = VOCAB - D


def _tc_transpose_body(x_hbm, t64_ref, o_ref, xa, xb, sem):
    j = pl.program_id(0)

    def load(slot, jj):
        return pltpu.make_async_copy(
            x_hbm.at[:, pl.ds(jj * _TCB, _TCB)], xa.at[slot], sem
        ), pltpu.make_async_copy(
            x_hbm.at[:, pl.ds(HV - _BSK + jj * _TCB, _BW)], xb.at[slot], sem
        )

    def load_tail(slot):
        return pltpu.make_async_copy(
            x_hbm.at[:, pl.ds((_NS - 1) * _TCB, 384)],
            xa.at[slot].at[:, pl.ds(0, 384)], sem
        ), pltpu.make_async_copy(
            x_hbm.at[:, pl.ds(_VA - 256, 256)],
            xb.at[slot].at[:, pl.ds(0, 256)], sem
        )

    def start(pair):
        pair[0].start()
        pair[1].start()

    def wait(pair):
        pair[0].wait()
        pair[1].wait()

    @pl.when(j == 0)
    def _prime():
        start(load(0, 0))

    @pl.when(j + 1 < _NS - 1)
    def _prefetch():
        start(load((j + 1) % 2, j + 1))

    @pl.when(j + 1 == _NS - 1)
    def _prefetch_tail():
        start(load_tail((j + 1) % 2))

    def xpose(x):
        return x.T

    @pl.when(j < _NS - 1)
    def _steady():
        wait(load(j % 2, j))
        bt = xpose(xb[j % 2])
        o_ref[...] = jnp.concatenate(
            [
                xpose(xa[j % 2]),
                lax.slice(bt, (_BSK, 0), (_BSK + _TCB, D)),
            ],
            axis=1,
        )

    @pl.when(j == _NS - 1)
    def _tail():
        wait(load_tail(j % 2))
        at = xpose(xa[j % 2, :, :384])
        bt = xpose(xb[j % 2, :, :256])
        t64t = xpose(t64_ref[...])
        bhalf = jnp.concatenate(
            [lax.slice(bt, (_BSK, 0), (256, D)), t64t], axis=0
        )
        o_ref[pl.ds(0, _TAIL), :] = jnp.concatenate(
            [lax.slice(at, (0, 0), (_TAIL, D)), bhalf], axis=1
        )


_tc_transpose = pl.pallas_call(
    _tc_transpose_body,
    grid=(_NS,),
    in_specs=[
        pl.BlockSpec(memory_space=pl.ANY),
        pl.BlockSpec((D, D), lambda j: (0, 0)),
    ],
    out_specs=pl.BlockSpec((_TCB, 2 * D), lambda j: (j, 0)),
    out_shape=jax.ShapeDtypeStruct((HV, 2 * D), jnp.float32),
    scratch_shapes=[
        pltpu.VMEM((2, D, _TCB), jnp.float32),
        pltpu.VMEM((2, D, _BW), jnp.float32),
        pltpu.SemaphoreType.DMA,
    ],
)


@functools.partial(
    pl.kernel,
    mesh=_mesh,
    compiler_params=pltpu.CompilerParams(use_tc_tiling_on_sc=False),
    out_type=jax.ShapeDtypeStruct((COLS, ROWS, DP), jnp.float32),
    scratch_types=[
        pltpu.VMEM((T_PER_W, RB // IW, IW), jnp.int32),
        pltpu.VMEM((2, C, D), jnp.float32),
        pltpu.SemaphoreType.DMA,
        pltpu.SemaphoreType.DMA,
    ],
)
def _gather_kernel(ids_hbm, table_hbm, out_hbm, idx_v, rows_v, sem_g, sem_s):
    wid = lax.axis_index("s") * NC + lax.axis_index("c")
    t0 = wid * T_PER_W

    for t in range(T_PER_W):
        pltpu.sync_copy(
            ids_hbm.at[pl.ds((t0 + t) * (RB // IW), RB // IW)], idx_v.at[t]
        )

    def remap(r, carry):
        row = idx_v.at[r // (RB // IW)].at[r % (RB // IW)]
        for u in range(IW // 16):
            x = row[pl.ds(u * 16, 16)]
            row[pl.ds(u * 16, 16)] = 2 * x - jnp.where(
                x >= HV, jnp.int32(2 * HV - 1), jnp.int32(0)
            )
        return carry

    lax.fori_loop(0, T_PER_W * (RB // IW), remap, 0)

    def fire_gathers(g, b):
        t = g // 2
        h = g % 2
        for j in range(KCH):
            pltpu.async_copy(
                table_hbm.at[idx_v.at[t].at[h * KCH + j]],
                rows_v.at[b].at[pl.ds(j * IW, IW)],
                sem_g,
            )

    def wait_gathers(b):
        pltpu.make_async_copy(
            out_hbm.at[0].at[pl.ds(0, C), pl.ds(0, D)], rows_v.at[b], sem_g
        ).wait()

    def store(g, b):
        task = t0 + g // 2
        col = task // (ROWS // RB)
        r_off = (task % (ROWS // RB)) * RB + (g % 2) * C
        return pltpu.async_copy(
            rows_v.at[b],
            out_hbm.at[col].at[pl.ds(r_off, C), pl.ds(0, D)],
            sem_s,
        )

    fire_gathers(0, 0)
    fire_gathers(1, 1)

    def body(p, carry):
        for b in range(2):
            g = 2 * p + b
            wait_gathers(b)
            store(g, b).wait()
            fire_gathers(g + 2, b)
        return carry

    lax.fori_loop(0, (NCH - 2) // 2, body, 0)

    for b in range(2):
        g = NCH - 2 + b
        wait_gathers(b)
        store(g, b).wait()


def kernel(ids, embs, pad):
    del pad
    ids_cm = ids.T.reshape(B // IW, IW).astype(jnp.int32)
    embs_t = embs.T
    table_rm = _tc_transpose(embs_t, embs_t[:, _VA:]).reshape(VOCAB, D)
    out_cm = _gather_kernel(ids_cm, table_rm)
    return out_cm.transpose(1, 0, 2)[:, :, :D]


# speedup vs baseline: 2.3797x; 1.2506x over previous
import functools

import jax
import jax.numpy as jnp
from jax import lax
from jax.experimental import pallas as pl
from jax.experimental.pallas import tpu as pltpu
from jax.experimental.pallas import tpu_sc as plsc

ROWS, COLS = 16384, 26
B = ROWS * COLS
D = 64
DP = 128
NC, NS = 2, 16
NW = NC * NS
IW = 128
RB = 1024
NTASK = COLS * (ROWS // RB)
T_PER_W = NTASK // NW
C = 512
KCH = C // IW
NCH = T_PER_W * (RB // C)

_mesh = plsc.VectorSubcoreMesh(core_axis_name="c", subcore_axis_name="s")

VOCAB = 1000000
HV = VOCAB // 2
_TCB = 2048


_NS = (HV + _TCB - 1) // _TCB
_TAIL = HV - (_NS - 1) * _TCB
_BW = _TCB + 128
_BSK = HV % 128
_VA = VOCAB - D


def _tc_transpose_body(x_hbm, t64_ref, o_ref, xa, xb, sem):
    j = pl.program_id(0)

    def load(slot, jj):
        return pltpu.make_async_copy(
            x_hbm.at[:, pl.ds(jj * _TCB, _TCB)], xa.at[slot], sem
        ), pltpu.make_async_copy(
            x_hbm.at[:, pl.ds(HV - _BSK + jj * _TCB, _BW)], xb.at[slot], sem
        )

    def load_tail(slot):
        return pltpu.make_async_copy(
            x_hbm.at[:, pl.ds((_NS - 1) * _TCB, 384)],
            xa.at[slot].at[:, pl.ds(0, 384)], sem
        ), pltpu.make_async_copy(
            x_hbm.at[:, pl.ds(_VA - 256, 256)],
            xb.at[slot].at[:, pl.ds(0, 256)], sem
        )

    def start(pair):
        pair[0].start()
        pair[1].start()

    def wait(pair):
        pair[0].wait()
        pair[1].wait()

    @pl.when(j == 0)
    def _prime():
        start(load(0, 0))

    @pl.when(j + 1 < _NS - 1)
    def _prefetch():
        start(load((j + 1) % 2, j + 1))

    @pl.when(j + 1 == _NS - 1)
    def _prefetch_tail():
        start(load_tail((j + 1) % 2))

    def xpose(x):
        return x.T

    @pl.when(j < _NS - 1)
    def _steady():
        wait(load(j % 2, j))
        bt = xpose(xb[j % 2])
        o_ref[...] = jnp.concatenate(
            [
                xpose(xa[j % 2]),
                lax.slice(bt, (_BSK, 0), (_BSK + _TCB, D)),
            ],
            axis=1,
        )

    @pl.when(j == _NS - 1)
    def _tail():
        wait(load_tail(j % 2))
        at = xpose(xa[j % 2, :, :384])
        bt = xpose(xb[j % 2, :, :256])
        t64t = xpose(t64_ref[...])
        bhalf = jnp.concatenate(
            [lax.slice(bt, (_BSK, 0), (256, D)), t64t], axis=0
        )
        o_ref[pl.ds(0, _TAIL), :] = jnp.concatenate(
            [lax.slice(at, (0, 0), (_TAIL, D)), bhalf], axis=1
        )


_tc_transpose = pl.pallas_call(
    _tc_transpose_body,
    grid=(_NS,),
    in_specs=[
        pl.BlockSpec(memory_space=pl.ANY),
        pl.BlockSpec((D, D), lambda j: (0, 0)),
    ],
    out_specs=pl.BlockSpec((_TCB, 2 * D), lambda j: (j, 0)),
    out_shape=jax.ShapeDtypeStruct((HV, 2 * D), jnp.float32),
    scratch_shapes=[
        pltpu.VMEM((2, D, _TCB), jnp.float32),
        pltpu.VMEM((2, D, _BW), jnp.float32),
        pltpu.SemaphoreType.DMA,
    ],
)


@functools.partial(
    pl.kernel,
    mesh=_mesh,
    compiler_params=pltpu.CompilerParams(use_tc_tiling_on_sc=False),
    out_type=jax.ShapeDtypeStruct((COLS, ROWS, DP), jnp.float32),
    scratch_types=[
        pltpu.VMEM((T_PER_W, RB // IW, IW), jnp.int32),
        pltpu.VMEM((2, C, D), jnp.float32),
        pltpu.SemaphoreType.DMA,
        pltpu.SemaphoreType.DMA,
    ],
)
def _gather_kernel(ids_hbm, table_hbm, out_hbm, idx_v, rows_v, sem_g, sem_s):
    wid = lax.axis_index("s") * NC + lax.axis_index("c")
    t0 = wid * T_PER_W

    for t in range(T_PER_W):
        pltpu.sync_copy(
            ids_hbm.at[pl.ds((t0 + t) * (RB // IW), RB // IW)], idx_v.at[t]
        )

    def remap(r, carry):
        row = idx_v.at[r // (RB // IW)].at[r % (RB // IW)]
        for u in range(IW // 16):
            x = row[pl.ds(u * 16, 16)]
            row[pl.ds(u * 16, 16)] = 2 * x - jnp.where(
                x >= HV, jnp.int32(2 * HV - 1), jnp.int32(0)
            )
        return carry

    lax.fori_loop(0, T_PER_W * (RB // IW), remap, 0)

    def fire_gathers(g, b):
        t = g // 2
        h = g % 2
        for j in range(KCH):
            pltpu.async_copy(
                table_hbm.at[idx_v.at[t].at[h * KCH + j]],
                rows_v.at[b].at[pl.ds(j * IW, IW)],
                sem_g,
            )

    def wait_gathers(b):
        pltpu.make_async_copy(
            out_hbm.at[0].at[pl.ds(0, C), pl.ds(0, D)], rows_v.at[b], sem_g
        ).wait()

    def store(g, b):
        task = t0 + g // 2
        col = task // (ROWS // RB)
        r_off = (task % (ROWS // RB)) * RB + (g % 2) * C
        return pltpu.async_copy(
            rows_v.at[b],
            out_hbm.at[col].at[pl.ds(r_off, C), pl.ds(0, D)],
            sem_s,
        )

    fire_gathers(0, 0)
    fire_gathers(1, 1)

    def body(p, carry):
        for b in range(2):
            g = 2 * p + b
            wait_gathers(b)
            store(g, b).wait()
            fire_gathers(g + 2, b)
        return carry

    lax.fori_loop(0, (NCH - 2) // 2, body, 0)

    for b in range(2):
        g = NCH - 2 + b
        wait_gathers(b)
        store(g, b).wait()


def kernel(ids, embs, pad):
    del pad
    ids_cm = ids.T.reshape(B // IW, IW).astype(jnp.int32)
    embs_t = embs.T
    table_rm = _tc_transpose(embs_t, embs_t[:, _VA:]).reshape(VOCAB, D)
    out_cm = _gather_kernel(ids_cm, table_rm)
    return out_cm.transpose(1, 0, 2)[:, :, :D]


# speedup vs baseline: 2.7646x; 1.1617x over previous
import functools

import jax
import jax.numpy as jnp
from jax import lax
from jax.experimental import pallas as pl
from jax.experimental.pallas import tpu as pltpu
from jax.experimental.pallas import tpu_sc as plsc

ROWS, COLS = 16384, 26
B = ROWS * COLS
D = 64
DP = 128
NC, NS = 2, 16
NW = NC * NS
IW = 128
RB = 1024
NTASK = COLS * (ROWS // RB)
T_PER_W = NTASK // NW
C = 512
KCH = C // IW
NCH = T_PER_W * (RB // C)

_mesh = plsc.VectorSubcoreMesh(core_axis_name="c", subcore_axis_name="s")

VOCAB = 1000000
HV = VOCAB // 2
_TCB = 4096


_NS = (HV + _TCB - 1) // _TCB
_TAIL = HV - (_NS - 1) * _TCB
_BW = _TCB + 128
_BSK = HV % 128
_VA = VOCAB - D


def _tc_transpose_body(x_hbm, t64_ref, o_ref, xa, xb, sem):
    j = pl.program_id(0)

    def load(slot, jj):
        return pltpu.make_async_copy(
            x_hbm.at[:, pl.ds(jj * _TCB, _TCB)], xa.at[slot], sem
        ), pltpu.make_async_copy(
            x_hbm.at[:, pl.ds(HV - _BSK + jj * _TCB, _BW)], xb.at[slot], sem
        )

    def load_tail(slot):
        return pltpu.make_async_copy(
            x_hbm.at[:, pl.ds((_NS - 1) * _TCB, 384)],
            xa.at[slot].at[:, pl.ds(0, 384)], sem
        ), pltpu.make_async_copy(
            x_hbm.at[:, pl.ds(_VA - 256, 256)],
            xb.at[slot].at[:, pl.ds(0, 256)], sem
        )

    def start(pair):
        pair[0].start()
        pair[1].start()

    def wait(pair):
        pair[0].wait()
        pair[1].wait()

    @pl.when(j == 0)
    def _prime():
        start(load(0, 0))

    @pl.when(j + 1 < _NS - 1)
    def _prefetch():
        start(load((j + 1) % 2, j + 1))

    @pl.when(j + 1 == _NS - 1)
    def _prefetch_tail():
        start(load_tail((j + 1) % 2))

    def xpose(x):
        return x.T

    @pl.when(j < _NS - 1)
    def _steady():
        wait(load(j % 2, j))
        bt = xpose(xb[j % 2])
        o_ref[...] = jnp.concatenate(
            [
                xpose(xa[j % 2]),
                lax.slice(bt, (_BSK, 0), (_BSK + _TCB, D)),
            ],
            axis=1,
        )

    @pl.when(j == _NS - 1)
    def _tail():
        wait(load_tail(j % 2))
        at = xpose(xa[j % 2, :, :384])
        bt = xpose(xb[j % 2, :, :256])
        t64t = xpose(t64_ref[...])
        bhalf = jnp.concatenate(
            [lax.slice(bt, (_BSK, 0), (256, D)), t64t], axis=0
        )
        o_ref[pl.ds(0, _TAIL), :] = jnp.concatenate(
            [lax.slice(at, (0, 0), (_TAIL, D)), bhalf], axis=1
        )


_tc_transpose = pl.pallas_call(
    _tc_transpose_body,
    grid=(_NS,),
    in_specs=[
        pl.BlockSpec(memory_space=pl.ANY),
        pl.BlockSpec((D, D), lambda j: (0, 0)),
    ],
    out_specs=pl.BlockSpec((_TCB, 2 * D), lambda j: (j, 0)),
    out_shape=jax.ShapeDtypeStruct((HV, 2 * D), jnp.float32),
    scratch_shapes=[
        pltpu.VMEM((2, D, _TCB), jnp.float32),
        pltpu.VMEM((2, D, _BW), jnp.float32),
        pltpu.SemaphoreType.DMA,
    ],
)


@functools.partial(
    pl.kernel,
    mesh=_mesh,
    compiler_params=pltpu.CompilerParams(use_tc_tiling_on_sc=False),
    out_type=jax.ShapeDtypeStruct((COLS, ROWS, DP), jnp.float32),
    scratch_types=[
        pltpu.VMEM((T_PER_W, RB // IW, IW), jnp.int32),
        pltpu.VMEM((2, C, D), jnp.float32),
        pltpu.SemaphoreType.DMA,
        pltpu.SemaphoreType.DMA,
    ],
)
def _gather_kernel(ids_hbm, table_hbm, out_hbm, idx_v, rows_v, sem_g, sem_s):
    wid = lax.axis_index("s") * NC + lax.axis_index("c")
    t0 = wid * T_PER_W

    for t in range(T_PER_W):
        pltpu.sync_copy(
            ids_hbm.at[pl.ds((t0 + t) * (RB // IW), RB // IW)], idx_v.at[t]
        )

    def remap(r, carry):
        row = idx_v.at[r // (RB // IW)].at[r % (RB // IW)]
        for u in range(IW // 16):
            x = row[pl.ds(u * 16, 16)]
            row[pl.ds(u * 16, 16)] = 2 * x - jnp.where(
                x >= HV, jnp.int32(2 * HV - 1), jnp.int32(0)
            )
        return carry

    lax.fori_loop(0, T_PER_W * (RB // IW), remap, 0)

    def fire_gathers(g, b):
        t = g // 2
        h = g % 2
        for j in range(KCH):
            pltpu.async_copy(
                table_hbm.at[idx_v.at[t].at[h * KCH + j]],
                rows_v.at[b].at[pl.ds(j * IW, IW)],
                sem_g,
            )

    def wait_gathers(b):
        pltpu.make_async_copy(
            out_hbm.at[0].at[pl.ds(0, C), pl.ds(0, D)], rows_v.at[b], sem_g
        ).wait()

    def store(g, b):
        task = t0 + g // 2
        col = task // (ROWS // RB)
        r_off = (task % (ROWS // RB)) * RB + (g % 2) * C
        return pltpu.async_copy(
            rows_v.at[b],
            out_hbm.at[col].at[pl.ds(r_off, C), pl.ds(0, D)],
            sem_s,
        )

    fire_gathers(0, 0)
    fire_gathers(1, 1)

    def body(p, carry):
        for b in range(2):
            g = 2 * p + b
            wait_gathers(b)
            store(g, b).wait()
            fire_gathers(g + 2, b)
        return carry

    lax.fori_loop(0, (NCH - 2) // 2, body, 0)

    for b in range(2):
        g = NCH - 2 + b
        wait_gathers(b)
        store(g, b).wait()


def kernel(ids, embs, pad):
    del pad
    ids_cm = ids.T.reshape(B // IW, IW).astype(jnp.int32)
    embs_t = embs.T
    table_rm = _tc_transpose(embs_t, embs_t[:, _VA:]).reshape(VOCAB, D)
    out_cm = _gather_kernel(ids_cm, table_rm)
    return out_cm.transpose(1, 0, 2)[:, :, :D]


# speedup vs baseline: 3.0033x; 1.0864x over previous
import functools

import jax
import jax.numpy as jnp
from jax import lax
from jax.experimental import pallas as pl
from jax.experimental.pallas import tpu as pltpu
from jax.experimental.pallas import tpu_sc as plsc

ROWS, COLS = 16384, 26
B = ROWS * COLS
D = 64
DP = 128
NC, NS = 2, 16
NW = NC * NS
IW = 128
RB = 1024
NTASK = COLS * (ROWS // RB)
T_PER_W = NTASK // NW
C = 512
KCH = C // IW
NCH = T_PER_W * (RB // C)

_mesh = plsc.VectorSubcoreMesh(core_axis_name="c", subcore_axis_name="s")

VOCAB = 1000000
HV = VOCAB // 2
_TCB = 8192


_NS = (HV + _TCB - 1) // _TCB
_TAIL = HV - (_NS - 1) * _TCB
_BW = _TCB + 128
_BSK = HV % 128
_VA = VOCAB - D


def _tc_transpose_body(x_hbm, t64_ref, o_ref, xa, xb, sem):
    j = pl.program_id(0)

    def load(slot, jj):
        return pltpu.make_async_copy(
            x_hbm.at[:, pl.ds(jj * _TCB, _TCB)], xa.at[slot], sem
        ), pltpu.make_async_copy(
            x_hbm.at[:, pl.ds(HV - _BSK + jj * _TCB, _BW)], xb.at[slot], sem
        )

    def load_tail(slot):
        return pltpu.make_async_copy(
            x_hbm.at[:, pl.ds((_NS - 1) * _TCB, 384)],
            xa.at[slot].at[:, pl.ds(0, 384)], sem
        ), pltpu.make_async_copy(
            x_hbm.at[:, pl.ds(_VA - 256, 256)],
            xb.at[slot].at[:, pl.ds(0, 256)], sem
        )

    def start(pair):
        pair[0].start()
        pair[1].start()

    def wait(pair):
        pair[0].wait()
        pair[1].wait()

    @pl.when(j == 0)
    def _prime():
        start(load(0, 0))

    @pl.when(j + 1 < _NS - 1)
    def _prefetch():
        start(load((j + 1) % 2, j + 1))

    @pl.when(j + 1 == _NS - 1)
    def _prefetch_tail():
        start(load_tail((j + 1) % 2))

    def xpose(x):
        return x.T

    @pl.when(j < _NS - 1)
    def _steady():
        wait(load(j % 2, j))
        bt = xpose(xb[j % 2])
        o_ref[...] = jnp.concatenate(
            [
                xpose(xa[j % 2]),
                lax.slice(bt, (_BSK, 0), (_BSK + _TCB, D)),
            ],
            axis=1,
        )

    @pl.when(j == _NS - 1)
    def _tail():
        wait(load_tail(j % 2))
        at = xpose(xa[j % 2, :, :384])
        bt = xpose(xb[j % 2, :, :256])
        t64t = xpose(t64_ref[...])
        bhalf = jnp.concatenate(
            [lax.slice(bt, (_BSK, 0), (256, D)), t64t], axis=0
        )
        o_ref[pl.ds(0, _TAIL), :] = jnp.concatenate(
            [lax.slice(at, (0, 0), (_TAIL, D)), bhalf], axis=1
        )


_tc_transpose = pl.pallas_call(
    _tc_transpose_body,
    grid=(_NS,),
    in_specs=[
        pl.BlockSpec(memory_space=pl.ANY),
        pl.BlockSpec((D, D), lambda j: (0, 0)),
    ],
    out_specs=pl.BlockSpec((_TCB, 2 * D), lambda j: (j, 0)),
    out_shape=jax.ShapeDtypeStruct((HV, 2 * D), jnp.float32),
    scratch_shapes=[
        pltpu.VMEM((2, D, _TCB), jnp.float32),
        pltpu.VMEM((2, D, _BW), jnp.float32),
        pltpu.SemaphoreType.DMA,
    ],
)


@functools.partial(
    pl.kernel,
    mesh=_mesh,
    compiler_params=pltpu.CompilerParams(use_tc_tiling_on_sc=False),
    out_type=jax.ShapeDtypeStruct((COLS, ROWS, DP), jnp.float32),
    scratch_types=[
        pltpu.VMEM((T_PER_W, RB // IW, IW), jnp.int32),
        pltpu.VMEM((2, C, D), jnp.float32),
        pltpu.SemaphoreType.DMA,
        pltpu.SemaphoreType.DMA,
    ],
)
def _gather_kernel(ids_hbm, table_hbm, out_hbm, idx_v, rows_v, sem_g, sem_s):
    wid = lax.axis_index("s") * NC + lax.axis_index("c")
    t0 = wid * T_PER_W

    for t in range(T_PER_W):
        pltpu.sync_copy(
            ids_hbm.at[pl.ds((t0 + t) * (RB // IW), RB // IW)], idx_v.at[t]
        )

    def remap(r, carry):
        row = idx_v.at[r // (RB // IW)].at[r % (RB // IW)]
        for u in range(IW // 16):
            x = row[pl.ds(u * 16, 16)]
            row[pl.ds(u * 16, 16)] = 2 * x - jnp.where(
                x >= HV, jnp.int32(2 * HV - 1), jnp.int32(0)
            )
        return carry

    lax.fori_loop(0, T_PER_W * (RB // IW), remap, 0)

    def fire_gathers(g, b):
        t = g // 2
        h = g % 2
        for j in range(KCH):
            pltpu.async_copy(
                table_hbm.at[idx_v.at[t].at[h * KCH + j]],
                rows_v.at[b].at[pl.ds(j * IW, IW)],
                sem_g,
            )

    def wait_gathers(b):
        pltpu.make_async_copy(
            out_hbm.at[0].at[pl.ds(0, C), pl.ds(0, D)], rows_v.at[b], sem_g
        ).wait()

    def store(g, b):
        task = t0 + g // 2
        col = task // (ROWS // RB)
        r_off = (task % (ROWS // RB)) * RB + (g % 2) * C
        return pltpu.async_copy(
            rows_v.at[b],
            out_hbm.at[col].at[pl.ds(r_off, C), pl.ds(0, D)],
            sem_s,
        )

    fire_gathers(0, 0)
    fire_gathers(1, 1)

    def body(p, carry):
        for b in range(2):
            g = 2 * p + b
            wait_gathers(b)
            store(g, b).wait()
            fire_gathers(g + 2, b)
        return carry

    lax.fori_loop(0, (NCH - 2) // 2, body, 0)

    for b in range(2):
        g = NCH - 2 + b
        wait_gathers(b)
        store(g, b).wait()


def kernel(ids, embs, pad):
    del pad
    ids_cm = ids.T.reshape(B // IW, IW).astype(jnp.int32)
    embs_t = embs.T
    table_rm = _tc_transpose(embs_t, embs_t[:, _VA:]).reshape(VOCAB, D)
    out_cm = _gather_kernel(ids_cm, table_rm)
    return out_cm.transpose(1, 0, 2)[:, :, :D]


# speedup vs baseline: 3.1280x; 1.0415x over previous
import functools

import jax
import jax.numpy as jnp
from jax import lax
from jax.experimental import pallas as pl
from jax.experimental.pallas import tpu as pltpu
from jax.experimental.pallas import tpu_sc as plsc

ROWS, COLS = 16384, 26
B = ROWS * COLS
D = 64
DP = 128
NC, NS = 2, 16
NW = NC * NS
IW = 128
RB = 1024
NTASK = COLS * (ROWS // RB)
T_PER_W = NTASK // NW
C = 512
KCH = C // IW
NCH = T_PER_W * (RB // C)

_mesh = plsc.VectorSubcoreMesh(core_axis_name="c", subcore_axis_name="s")

VOCAB = 1000000
HV = VOCAB // 2
_TCB = 16384


_NS = (HV + _TCB - 1) // _TCB
_TAIL = HV - (_NS - 1) * _TCB
_BW = _TCB + 128
_BSK = HV % 128
_VA = VOCAB - D
_TAW = -(-_TAIL // 128) * 128
_BV = _VA - (HV + (_NS - 1) * _TCB)
_TBW = -(-_BV // 128) * 128


def _tc_transpose_body(x_hbm, t64_ref, o_ref, xa, xb, sem):
    j = pl.program_id(0)

    def load(slot, jj):
        return pltpu.make_async_copy(
            x_hbm.at[:, pl.ds(jj * _TCB, _TCB)], xa.at[slot], sem
        ), pltpu.make_async_copy(
            x_hbm.at[:, pl.ds(HV - _BSK + jj * _TCB, _BW)], xb.at[slot], sem
        )

    def load_tail(slot):
        return pltpu.make_async_copy(
            x_hbm.at[:, pl.ds((_NS - 1) * _TCB, _TAW)],
            xa.at[slot].at[:, pl.ds(0, _TAW)], sem
        ), pltpu.make_async_copy(
            x_hbm.at[:, pl.ds(_VA - _TBW, _TBW)],
            xb.at[slot].at[:, pl.ds(0, _TBW)], sem
        )

    def start(pair):
        pair[0].start()
        pair[1].start()

    def wait(pair):
        pair[0].wait()
        pair[1].wait()

    @pl.when(j == 0)
    def _prime():
        start(load(0, 0))

    @pl.when(j + 1 < _NS - 1)
    def _prefetch():
        start(load((j + 1) % 2, j + 1))

    @pl.when(j + 1 == _NS - 1)
    def _prefetch_tail():
        start(load_tail((j + 1) % 2))

    def xpose(x):
        return x.T

    @pl.when(j < _NS - 1)
    def _steady():
        wait(load(j % 2, j))
        bt = xpose(xb[j % 2])
        o_ref[...] = jnp.concatenate(
            [
                xpose(xa[j % 2]),
                lax.slice(bt, (_BSK, 0), (_BSK + _TCB, D)),
            ],
            axis=1,
        )

    @pl.when(j == _NS - 1)
    def _tail():
        wait(load_tail(j % 2))
        at = xpose(xa[j % 2, :, :_TAW])
        bt = xpose(xb[j % 2, :, :_TBW])
        t64t = xpose(t64_ref[...])
        bhalf = jnp.concatenate(
            [lax.slice(bt, (_TBW - _BV, 0), (_TBW, D)), t64t], axis=0
        )
        o_ref[pl.ds(0, _TAIL), :] = jnp.concatenate(
            [lax.slice(at, (0, 0), (_TAIL, D)), bhalf], axis=1
        )


_tc_transpose = pl.pallas_call(
    _tc_transpose_body,
    grid=(_NS,),
    in_specs=[
        pl.BlockSpec(memory_space=pl.ANY),
        pl.BlockSpec((D, D), lambda j: (0, 0)),
    ],
    out_specs=pl.BlockSpec((_TCB, 2 * D), lambda j: (j, 0)),
    out_shape=jax.ShapeDtypeStruct((HV, 2 * D), jnp.float32),
    scratch_shapes=[
        pltpu.VMEM((2, D, _TCB), jnp.float32),
        pltpu.VMEM((2, D, _BW), jnp.float32),
        pltpu.SemaphoreType.DMA,
    ],
)


@functools.partial(
    pl.kernel,
    mesh=_mesh,
    compiler_params=pltpu.CompilerParams(use_tc_tiling_on_sc=False),
    out_type=jax.ShapeDtypeStruct((COLS, ROWS, DP), jnp.float32),
    scratch_types=[
        pltpu.VMEM((T_PER_W, RB // IW, IW), jnp.int32),
        pltpu.VMEM((2, C, D), jnp.float32),
        pltpu.SemaphoreType.DMA,
        pltpu.SemaphoreType.DMA,
    ],
)
def _gather_kernel(ids_hbm, table_hbm, out_hbm, idx_v, rows_v, sem_g, sem_s):
    wid = lax.axis_index("s") * NC + lax.axis_index("c")
    t0 = wid * T_PER_W

    for t in range(T_PER_W):
        pltpu.sync_copy(
            ids_hbm.at[pl.ds((t0 + t) * (RB // IW), RB // IW)], idx_v.at[t]
        )

    def remap(r, carry):
        row = idx_v.at[r // (RB // IW)].at[r % (RB // IW)]
        for u in range(IW // 16):
            x = row[pl.ds(u * 16, 16)]
            row[pl.ds(u * 16, 16)] = 2 * x - jnp.where(
                x >= HV, jnp.int32(2 * HV - 1), jnp.int32(0)
            )
        return carry

    lax.fori_loop(0, T_PER_W * (RB // IW), remap, 0)

    def fire_gathers(g, b):
        t = g // 2
        h = g % 2
        for j in range(KCH):
            pltpu.async_copy(
                table_hbm.at[idx_v.at[t].at[h * KCH + j]],
                rows_v.at[b].at[pl.ds(j * IW, IW)],
                sem_g,
            )

    def wait_gathers(b):
        pltpu.make_async_copy(
            out_hbm.at[0].at[pl.ds(0, C), pl.ds(0, D)], rows_v.at[b], sem_g
        ).wait()

    def store(g, b):
        task = t0 + g // 2
        col = task // (ROWS // RB)
        r_off = (task % (ROWS // RB)) * RB + (g % 2) * C
        return pltpu.async_copy(
            rows_v.at[b],
            out_hbm.at[col].at[pl.ds(r_off, C), pl.ds(0, D)],
            sem_s,
        )

    fire_gathers(0, 0)
    fire_gathers(1, 1)

    def body(p, carry):
        for b in range(2):
            g = 2 * p + b
            wait_gathers(b)
            store(g, b).wait()
            fire_gathers(g + 2, b)
        return carry

    lax.fori_loop(0, (NCH - 2) // 2, body, 0)

    for b in range(2):
        g = NCH - 2 + b
        wait_gathers(b)
        store(g, b).wait()


def kernel(ids, embs, pad):
    del pad
    ids_cm = ids.T.reshape(B // IW, IW).astype(jnp.int32)
    embs_t = embs.T
    table_rm = _tc_transpose(embs_t, embs_t[:, _VA:]).reshape(VOCAB, D)
    out_cm = _gather_kernel(ids_cm, table_rm)
    return out_cm.transpose(1, 0, 2)[:, :, :D]
